# baseline (speedup 1.0000x reference)
import functools

import jax
import jax.numpy as jnp
from jax import lax
from jax.experimental import pallas as pl
from jax.experimental.pallas import tpu as pltpu
from jax.experimental.pallas import tpu_sc as plsc

F32 = jnp.float32
BF16 = jnp.bfloat16
I32 = jnp.int32

T = 4096
H = 1024
E = 8
CAP = 512
B = 2
S = 2048
DISP_ROWS = 4608
TRASH = 4096
NW = 32
RPW = T // NW

_BLK1 = 512
_BLKR = 256
_BLKF = 256


def _stage1_body(x_ref, w_ref, b_ref, wg_ref, hid_ref, log_ref):
    xb = x_ref[...].astype(BF16)
    h = jnp.dot(xb, w_ref[...].astype(BF16), preferred_element_type=F32)
    h = h + b_ref[...]
    hid_ref[...] = h.astype(BF16)
    log_ref[...] = jnp.dot(h.astype(BF16), wg_ref[...].astype(BF16),
                           preferred_element_type=F32)


def _routing_body(l_ref, ss_ref, cs_ref, g_ref, carry_ref):
    pid = pl.program_id(0)

    @pl.when(pid == 0)
    def _():
        carry_ref[...] = jnp.zeros_like(carry_ref)

    l = l_ref[...]
    n = l.shape[0]
    m = jnp.max(l, axis=1, keepdims=True)
    s = jnp.sum(jnp.exp(l - m), axis=1, keepdims=True)
    gv = 1.0 / s
    ei = lax.broadcasted_iota(I32, (n, E), 1)
    idx = jnp.min(jnp.where(l == m, ei, E), axis=1, keepdims=True)
    mask = (ei == idx).astype(F32)
    ri = lax.broadcasted_iota(I32, (n, n), 0)
    ci = lax.broadcasted_iota(I32, (n, n), 1)
    tril = (ci <= ri).astype(BF16)
    incl = jnp.dot(tril, mask.astype(BF16), preferred_element_type=F32)
    incl = jnp.sum(incl * mask, axis=1, keepdims=True)
    carry = carry_ref[...]
    base = jnp.sum(carry * mask, axis=1, keepdims=True)
    carry_ref[...] = carry + jnp.sum(mask, axis=0, keepdims=True)
    loc = base + incl - 1.0
    keep = loc < CAP
    locc = jnp.minimum(loc, CAP - 1.0).astype(I32)
    slot = idx * CAP + locc
    ss_ref[...] = jnp.where(keep, slot, TRASH)
    cs_ref[...] = slot
    g_ref[...] = jnp.where(keep, gv, 0.0)


def _ffn_body(d_ref, wa_ref, ba_ref, wb_ref, bb_ref, h_ref):
    lhs = d_ref[...]
    t = jnp.dot(lhs, wa_ref[0].astype(BF16), preferred_element_type=F32)
    t = t + ba_ref[0]
    h = jnp.dot(t.astype(BF16), wb_ref[0].astype(BF16),
                preferred_element_type=F32)
    h = h + bb_ref[0]
    h_ref[...] = h.astype(BF16)


def _w2_body(r_ref, g_ref, w_ref, b_ref, wg_ref, o_ref, log_ref):
    lhs = (r_ref[...].astype(F32) * g_ref[...]).astype(BF16)
    o = jnp.dot(lhs, w_ref[...].astype(BF16), preferred_element_type=F32)
    o = o + b_ref[...]
    o_ref[...] = o.astype(BF16)
    log_ref[...] = jnp.dot(o.astype(BF16), wg_ref[...].astype(BF16),
                           preferred_element_type=F32)


def _final_body(h_ref, r_ref, g_ref, w3_ref, b3_ref, y_ref, out_ref,
                acch_ref, acco_ref):
    pid = pl.program_id(0)

    @pl.when(pid == 0)
    def _():
        acch_ref[...] = jnp.zeros_like(acch_ref)
        acco_ref[...] = jnp.zeros_like(acco_ref)
        out_ref[...] = jnp.zeros_like(out_ref)

    b = pid // (S // _BLKF)
    rowsel = (lax.broadcasted_iota(I32, (B, 1), 0) == b).astype(F32)
    hsum = jnp.sum(h_ref[...].astype(F32), axis=0, keepdims=True)
    osum = jnp.sum(r_ref[...].astype(F32) * g_ref[...], axis=0, keepdims=True)
    acch_ref[...] += rowsel * hsum
    acco_ref[...] += rowsel * osum

    @pl.when(pid == pl.num_programs(0) - 1)
    def _():
        sent = acch_ref[...] * (1.0 / S)
        sent = sent + jnp.dot((acco_ref[...] * (1.0 / S)).astype(BF16),
                              w3_ref[...].astype(BF16),
                              preferred_element_type=F32)
        sent = sent + b3_ref[...]
        m = jnp.max(sent, axis=1, keepdims=True)
        lz = jnp.log(jnp.sum(jnp.exp(sent - m), axis=1, keepdims=True)) + m
        ci = lax.broadcasted_iota(I32, (B, H), 1)
        picked = jnp.sum(jnp.where(ci == y_ref[...], sent, 0.0),
                         axis=1, keepdims=True)
        out_ref[...] = jnp.sum(lz - picked, axis=0, keepdims=True) / B


def _stage1(xf, W1, b1, Wg1):
    grid = (T // _BLK1,)
    return pl.pallas_call(
        _stage1_body,
        grid=grid,
        in_specs=[
            pl.BlockSpec((_BLK1, H), lambda i: (i, 0)),
            pl.BlockSpec((H, H), lambda i: (0, 0)),
            pl.BlockSpec((1, H), lambda i: (0, 0)),
            pl.BlockSpec((H, E), lambda i: (0, 0)),
        ],
        out_specs=[
            pl.BlockSpec((_BLK1, H), lambda i: (i, 0)),
            pl.BlockSpec((_BLK1, E), lambda i: (i, 0)),
        ],
        out_shape=[
            jax.ShapeDtypeStruct((T, H), BF16),
            jax.ShapeDtypeStruct((T, E), F32),
        ],
        compiler_params=pltpu.CompilerParams(
            dimension_semantics=("arbitrary",)),
    )(xf, W1, b1, Wg1)


def _routing(logits):
    grid = (T // _BLKR,)
    return pl.pallas_call(
        _routing_body,
        grid=grid,
        in_specs=[pl.BlockSpec((_BLKR, E), lambda i: (i, 0))],
        out_specs=[
            pl.BlockSpec((_BLKR, 1), lambda i: (i, 0)),
            pl.BlockSpec((_BLKR, 1), lambda i: (i, 0)),
            pl.BlockSpec((_BLKR, 1), lambda i: (i, 0)),
        ],
        out_shape=[
            jax.ShapeDtypeStruct((T, 1), I32),
            jax.ShapeDtypeStruct((T, 1), I32),
            jax.ShapeDtypeStruct((T, 1), F32),
        ],
        scratch_shapes=[pltpu.VMEM((1, E), F32)],
        compiler_params=pltpu.CompilerParams(
            dimension_semantics=("arbitrary",)),
    )(logits)


def _ffn(disp, Wa, ba, Wb, bb):
    grid = (E,)
    return pl.pallas_call(
        _ffn_body,
        grid=grid,
        in_specs=[
            pl.BlockSpec((CAP, H), lambda e: (e, 0)),
            pl.BlockSpec((1, H, H), lambda e: (e, 0, 0)),
            pl.BlockSpec((1, 1, H), lambda e: (e, 0, 0)),
            pl.BlockSpec((1, H, H), lambda e: (e, 0, 0)),
            pl.BlockSpec((1, 1, H), lambda e: (e, 0, 0)),
        ],
        out_specs=[pl.BlockSpec((CAP, H), lambda e: (e, 0))],
        out_shape=[jax.ShapeDtypeStruct((T, H), BF16)],
        compiler_params=pltpu.CompilerParams(
            dimension_semantics=("arbitrary",)),
    )(disp, Wa, ba, Wb, bb)[0]


def _w2(rows, gate, W2, b2, Wg2):
    grid = (T // _BLK1,)
    return pl.pallas_call(
        _w2_body,
        grid=grid,
        in_specs=[
            pl.BlockSpec((_BLK1, H), lambda i: (i, 0)),
            pl.BlockSpec((_BLK1, 1), lambda i: (i, 0)),
            pl.BlockSpec((H, H), lambda i: (0, 0)),
            pl.BlockSpec((1, H), lambda i: (0, 0)),
            pl.BlockSpec((H, E), lambda i: (0, 0)),
        ],
        out_specs=[
            pl.BlockSpec((_BLK1, H), lambda i: (i, 0)),
            pl.BlockSpec((_BLK1, E), lambda i: (i, 0)),
        ],
        out_shape=[
            jax.ShapeDtypeStruct((T, H), BF16),
            jax.ShapeDtypeStruct((T, E), F32),
        ],
        compiler_params=pltpu.CompilerParams(
            dimension_semantics=("arbitrary",)),
    )(rows, gate, W2, b2, Wg2)


def _final(hidden, rows2, gate2, W3, b3, y2):
    grid = (T // _BLKF,)
    return pl.pallas_call(
        _final_body,
        grid=grid,
        in_specs=[
            pl.BlockSpec((_BLKF, H), lambda i: (i, 0)),
            pl.BlockSpec((_BLKF, H), lambda i: (i, 0)),
            pl.BlockSpec((_BLKF, 1), lambda i: (i, 0)),
            pl.BlockSpec((H, H), lambda i: (0, 0)),
            pl.BlockSpec((1, H), lambda i: (0, 0)),
            pl.BlockSpec((B, 1), lambda i: (0, 0)),
        ],
        out_specs=[pl.BlockSpec((1, 1), lambda i: (0, 0))],
        out_shape=[jax.ShapeDtypeStruct((1, 1), F32)],
        scratch_shapes=[pltpu.VMEM((B, H), F32), pltpu.VMEM((B, H), F32)],
        compiler_params=pltpu.CompilerParams(
            dimension_semantics=("arbitrary",)),
    )(hidden, rows2, gate2, W3, b3, y2)[0]


def _sc_mesh():
    return plsc.VectorSubcoreMesh(core_axis_name="c", subcore_axis_name="s")


def _rows_to_i32(a, n):
    return lax.bitcast_convert_type(a.reshape(n, H // 2, 2),
                                    I32).reshape(n, H // 256, 128)


def _rows_from_i32(a, n):
    return lax.bitcast_convert_type(a.reshape(n, H // 2),
                                    BF16).reshape(n, H)


def _sc_scatter(src3, slots):
    @functools.partial(
        pl.kernel,
        out_type=jax.ShapeDtypeStruct((DISP_ROWS, H // 256, 128), I32),
        mesh=_sc_mesh(),
        scratch_types=[
            pltpu.VMEM((RPW,), I32),
            pltpu.VMEM((RPW, H // 256, 128), I32),
            pltpu.SemaphoreType.DMA,
        ],
    )
    def k(src_hbm, slot_hbm, out_hbm, idx_v, rows_v, sem):
        wid = lax.axis_index("s") * 2 + lax.axis_index("c")
        base = wid * RPW
        pltpu.sync_copy(slot_hbm.at[pl.ds(base, RPW)], idx_v)
        pltpu.sync_copy(src_hbm.at[pl.ds(base, RPW)], rows_v)
        pltpu.async_copy(rows_v, out_hbm.at[idx_v], sem).wait()

    return k(src3, slots)


def _sc_gather(src3, slots):
    @functools.partial(
        pl.kernel,
        out_type=jax.ShapeDtypeStruct((T, H // 256, 128), I32),
        mesh=_sc_mesh(),
        scratch_types=[
            pltpu.VMEM((RPW,), I32),
            pltpu.VMEM((RPW, H // 256, 128), I32),
            pltpu.SemaphoreType.DMA,
        ],
    )
    def k(src_hbm, slot_hbm, out_hbm, idx_v, rows_v, sem):
        wid = lax.axis_index("s") * 2 + lax.axis_index("c")
        base = wid * RPW
        pltpu.sync_copy(slot_hbm.at[pl.ds(base, RPW)], idx_v)
        pltpu.async_copy(src_hbm.at[idx_v], rows_v, sem).wait()
        pltpu.sync_copy(rows_v, out_hbm.at[pl.ds(base, RPW)])

    return k(src3, slots)


def kernel(x, y, W1, b1, Wg1, We1a, be1a, We1b, be1b, W2, b2, Wg2,
           We2a, be2a, We2b, be2b, W3, b3):
    xf = x.reshape(T, H)
    hidden16, logits1 = _stage1(xf, W1, b1.reshape(1, H), Wg1)

    ss1, cs1, gate1 = _routing(logits1)
    disp1 = _sc_scatter(_rows_to_i32(hidden16, T), ss1.reshape(T))
    h1 = _ffn(_rows_from_i32(disp1, DISP_ROWS), We1a, be1a.reshape(E, 1, H),
              We1b, be1b.reshape(E, 1, H))
    rows1 = _sc_gather(_rows_to_i32(h1, T), cs1.reshape(T))

    out16, logits2 = _w2(_rows_from_i32(rows1, T), gate1, W2,
                         b2.reshape(1, H), Wg2)

    ss2, cs2, gate2 = _routing(logits2)
    disp2 = _sc_scatter(_rows_to_i32(out16, T), ss2.reshape(T))
    h2 = _ffn(_rows_from_i32(disp2, DISP_ROWS), We2a, be2a.reshape(E, 1, H),
              We2b, be2b.reshape(E, 1, H))
    rows2 = _sc_gather(_rows_to_i32(h2, T), cs2.reshape(T))

    loss = _final(hidden16, _rows_from_i32(rows2, T), gate2, W3,
                  b3.reshape(1, H), y.reshape(B, 1).astype(I32))
    return loss.reshape(())


# speedup vs baseline: 2.4408x; 2.4408x over previous
import functools

import jax
import jax.numpy as jnp
from jax import lax
from jax.experimental import pallas as pl
from jax.experimental.pallas import tpu as pltpu
from jax.experimental.pallas import tpu_sc as plsc

F32 = jnp.float32
BF16 = jnp.bfloat16
I32 = jnp.int32

T = 4096
H = 1024
E = 8
CAP = 512
B = 2
S = 2048
DISP_ROWS = 4608
TRASH = 4096
NW = 32
RPW = T // NW
GW = 32
NCH = RPW // GW
NBUF = 3

_BLK1 = 512
_BLKR = 256
_BLKF = 256


def _stage1_body(x_ref, w_ref, b_ref, wg_ref, hid_ref, log_ref):
    xb = x_ref[...].astype(BF16)
    h = jnp.dot(xb, w_ref[...].astype(BF16), preferred_element_type=F32)
    h = h + b_ref[...]
    hid_ref[...] = h
    log_ref[...] = jnp.dot(h.astype(BF16), wg_ref[...].astype(BF16),
                           preferred_element_type=F32)


def _routing_body(l_ref, ss_ref, cs_ref, g_ref, carry_ref):
    pid = pl.program_id(0)

    @pl.when(pid == 0)
    def _():
        carry_ref[...] = jnp.zeros_like(carry_ref)

    l = l_ref[...]
    n = l.shape[0]
    m = jnp.max(l, axis=1, keepdims=True)
    s = jnp.sum(jnp.exp(l - m), axis=1, keepdims=True)
    gv = 1.0 / s
    ei = lax.broadcasted_iota(I32, (n, E), 1)
    idx = jnp.min(jnp.where(l == m, ei, E), axis=1, keepdims=True)
    mask = (ei == idx).astype(F32)
    ri = lax.broadcasted_iota(I32, (n, n), 0)
    ci = lax.broadcasted_iota(I32, (n, n), 1)
    tril = (ci <= ri).astype(BF16)
    incl = jnp.dot(tril, mask.astype(BF16), preferred_element_type=F32)
    incl = jnp.sum(incl * mask, axis=1, keepdims=True)
    carry = carry_ref[...]
    base = jnp.sum(carry * mask, axis=1, keepdims=True)
    carry_ref[...] = carry + jnp.sum(mask, axis=0, keepdims=True)
    loc = base + incl - 1.0
    keep = loc < CAP
    locc = jnp.minimum(loc, CAP - 1.0).astype(I32)
    slot = idx * CAP + locc
    ss_ref[...] = jnp.where(keep, slot, TRASH)
    cs_ref[...] = slot
    g_ref[...] = jnp.where(keep, gv, 0.0)


def _ffn_body(d_ref, wa_ref, ba_ref, wb_ref, bb_ref, h_ref):
    lhs = d_ref[...].astype(BF16)
    t = jnp.dot(lhs, wa_ref[0].astype(BF16), preferred_element_type=F32)
    t = t + ba_ref[0]
    h = jnp.dot(t.astype(BF16), wb_ref[0].astype(BF16),
                preferred_element_type=F32)
    h = h + bb_ref[0]
    h_ref[...] = h


def _w2_body(r_ref, g_ref, w_ref, b_ref, wg_ref, o_ref, log_ref):
    lhs = (r_ref[...] * g_ref[...]).astype(BF16)
    o = jnp.dot(lhs, w_ref[...].astype(BF16), preferred_element_type=F32)
    o = o + b_ref[...]
    o_ref[...] = o
    log_ref[...] = jnp.dot(o.astype(BF16), wg_ref[...].astype(BF16),
                           preferred_element_type=F32)


def _final_body(h_ref, r_ref, g_ref, w3_ref, b3_ref, y_ref, out_ref,
                acch_ref, acco_ref):
    pid = pl.program_id(0)

    @pl.when(pid == 0)
    def _():
        acch_ref[...] = jnp.zeros_like(acch_ref)
        acco_ref[...] = jnp.zeros_like(acco_ref)
        out_ref[...] = jnp.zeros_like(out_ref)

    b = pid // (S // _BLKF)
    rowsel = (lax.broadcasted_iota(I32, (B, 1), 0) == b).astype(F32)
    hsum = jnp.sum(h_ref[...], axis=0, keepdims=True)
    osum = jnp.sum(r_ref[...] * g_ref[...], axis=0, keepdims=True)
    acch_ref[...] += rowsel * hsum
    acco_ref[...] += rowsel * osum

    @pl.when(pid == pl.num_programs(0) - 1)
    def _():
        sent = acch_ref[...] * (1.0 / S)
        sent = sent + jnp.dot((acco_ref[...] * (1.0 / S)).astype(BF16),
                              w3_ref[...].astype(BF16),
                              preferred_element_type=F32)
        sent = sent + b3_ref[...]
        m = jnp.max(sent, axis=1, keepdims=True)
        lz = jnp.log(jnp.sum(jnp.exp(sent - m), axis=1, keepdims=True)) + m
        ci = lax.broadcasted_iota(I32, (B, H), 1)
        picked = jnp.sum(jnp.where(ci == y_ref[...], sent, 0.0),
                         axis=1, keepdims=True)
        out_ref[...] = jnp.sum(lz - picked, axis=0, keepdims=True) / B


def _stage1(xf, W1, b1, Wg1):
    grid = (T // _BLK1,)
    return pl.pallas_call(
        _stage1_body,
        grid=grid,
        in_specs=[
            pl.BlockSpec((_BLK1, H), lambda i: (i, 0)),
            pl.BlockSpec((H, H), lambda i: (0, 0)),
            pl.BlockSpec((1, H), lambda i: (0, 0)),
            pl.BlockSpec((H, E), lambda i: (0, 0)),
        ],
        out_specs=[
            pl.BlockSpec((_BLK1, H), lambda i: (i, 0)),
            pl.BlockSpec((_BLK1, E), lambda i: (i, 0)),
        ],
        out_shape=[
            jax.ShapeDtypeStruct((T, H), F32),
            jax.ShapeDtypeStruct((T, E), F32),
        ],
        compiler_params=pltpu.CompilerParams(
            dimension_semantics=("arbitrary",)),
    )(xf, W1, b1, Wg1)


def _routing(logits):
    grid = (T // _BLKR,)
    return pl.pallas_call(
        _routing_body,
        grid=grid,
        in_specs=[pl.BlockSpec((_BLKR, E), lambda i: (i, 0))],
        out_specs=[
            pl.BlockSpec((_BLKR, 1), lambda i: (i, 0)),
            pl.BlockSpec((_BLKR, 1), lambda i: (i, 0)),
            pl.BlockSpec((_BLKR, 1), lambda i: (i, 0)),
        ],
        out_shape=[
            jax.ShapeDtypeStruct((T, 1), I32),
            jax.ShapeDtypeStruct((T, 1), I32),
            jax.ShapeDtypeStruct((T, 1), F32),
        ],
        scratch_shapes=[pltpu.VMEM((1, E), F32)],
        compiler_params=pltpu.CompilerParams(
            dimension_semantics=("arbitrary",)),
    )(logits)


def _ffn(disp, Wa, ba, Wb, bb):
    grid = (E,)
    return pl.pallas_call(
        _ffn_body,
        grid=grid,
        in_specs=[
            pl.BlockSpec((CAP, H), lambda e: (e, 0)),
            pl.BlockSpec((1, H, H), lambda e: (e, 0, 0)),
            pl.BlockSpec((1, 1, H), lambda e: (e, 0, 0)),
            pl.BlockSpec((1, H, H), lambda e: (e, 0, 0)),
            pl.BlockSpec((1, 1, H), lambda e: (e, 0, 0)),
        ],
        out_specs=[pl.BlockSpec((CAP, H), lambda e: (e, 0))],
        out_shape=[jax.ShapeDtypeStruct((T, H), F32)],
        compiler_params=pltpu.CompilerParams(
            dimension_semantics=("arbitrary",)),
    )(disp, Wa, ba, Wb, bb)[0]


def _w2(rows, gate, W2, b2, Wg2):
    grid = (T // _BLK1,)
    return pl.pallas_call(
        _w2_body,
        grid=grid,
        in_specs=[
            pl.BlockSpec((_BLK1, H), lambda i: (i, 0)),
            pl.BlockSpec((_BLK1, 1), lambda i: (i, 0)),
            pl.BlockSpec((H, H), lambda i: (0, 0)),
            pl.BlockSpec((1, H), lambda i: (0, 0)),
            pl.BlockSpec((H, E), lambda i: (0, 0)),
        ],
        out_specs=[
            pl.BlockSpec((_BLK1, H), lambda i: (i, 0)),
            pl.BlockSpec((_BLK1, E), lambda i: (i, 0)),
        ],
        out_shape=[
            jax.ShapeDtypeStruct((T, H), F32),
            jax.ShapeDtypeStruct((T, E), F32),
        ],
        compiler_params=pltpu.CompilerParams(
            dimension_semantics=("arbitrary",)),
    )(rows, gate, W2, b2, Wg2)


def _final(hidden, rows2, gate2, W3, b3, y2):
    grid = (T // _BLKF,)
    return pl.pallas_call(
        _final_body,
        grid=grid,
        in_specs=[
            pl.BlockSpec((_BLKF, H), lambda i: (i, 0)),
            pl.BlockSpec((_BLKF, H), lambda i: (i, 0)),
            pl.BlockSpec((_BLKF, 1), lambda i: (i, 0)),
            pl.BlockSpec((H, H), lambda i: (0, 0)),
            pl.BlockSpec((1, H), lambda i: (0, 0)),
            pl.BlockSpec((B, 1), lambda i: (0, 0)),
        ],
        out_specs=[pl.BlockSpec((1, 1), lambda i: (0, 0))],
        out_shape=[jax.ShapeDtypeStruct((1, 1), F32)],
        scratch_shapes=[pltpu.VMEM((B, H), F32), pltpu.VMEM((B, H), F32)],
        compiler_params=pltpu.CompilerParams(
            dimension_semantics=("arbitrary",)),
    )(hidden, rows2, gate2, W3, b3, y2)[0]


def _sc_mesh():
    return plsc.VectorSubcoreMesh(core_axis_name="c", subcore_axis_name="s")


_SC_SCRATCH = [
    pltpu.VMEM((NCH, GW), I32),
    pltpu.VMEM((NBUF, GW, H), F32),
] + [pltpu.SemaphoreType.DMA] * (2 * NBUF)


def _sc_move(src, slots2, out_rows, indirect_out):
    @functools.partial(
        pl.kernel,
        out_type=jax.ShapeDtypeStruct((out_rows, H), F32),
        mesh=_sc_mesh(),
        scratch_types=_SC_SCRATCH,
    )
    def k(src_hbm, slot_hbm, out_hbm, idx_v, buf, *sems):
        sin, sout = sems[:NBUF], sems[NBUF:]
        wid = lax.axis_index("s") * 2 + lax.axis_index("c")
        base = wid * RPW
        pltpu.sync_copy(slot_hbm.at[pl.ds(wid * NCH, NCH)], idx_v)

        def start_in(j):
            b = j % NBUF
            if indirect_out:
                return pltpu.async_copy(
                    src_hbm.at[pl.ds(base + j * GW, GW)], buf.at[b], sin[b])
            return pltpu.async_copy(
                src_hbm.at[idx_v.at[j]], buf.at[b], sin[b])

        def start_out(j):
            b = j % NBUF
            if indirect_out:
                return pltpu.async_copy(
                    buf.at[b], out_hbm.at[idx_v.at[j]], sout[b])
            return pltpu.async_copy(
                buf.at[b], out_hbm.at[pl.ds(base + j * GW, GW)], sout[b])

        ins = [None] * NCH
        outs = [None] * NCH
        for j in range(min(NBUF, NCH)):
            ins[j] = start_in(j)
        for j in range(NCH):
            ins[j].wait()
            outs[j] = start_out(j)
            nxt = j + NBUF
            if nxt < NCH:
                outs[j].wait()
                ins[nxt] = start_in(nxt)
        for j in range(max(0, NCH - NBUF), NCH):
            outs[j].wait()

    return k(src, slots2)


def _sc_scatter(src, slots2):
    return _sc_move(src, slots2, DISP_ROWS, True)


def _sc_gather(src, slots2):
    return _sc_move(src, slots2, T, False)


def kernel(x, y, W1, b1, Wg1, We1a, be1a, We1b, be1b, W2, b2, Wg2,
           We2a, be2a, We2b, be2b, W3, b3):
    xf = x.reshape(T, H)
    hidden16, logits1 = _stage1(xf, W1, b1.reshape(1, H), Wg1)

    ss1, cs1, gate1 = _routing(logits1)
    disp1 = _sc_scatter(hidden16, ss1.reshape(NW * NCH, GW))
    h1 = _ffn(disp1, We1a, be1a.reshape(E, 1, H),
              We1b, be1b.reshape(E, 1, H))
    rows1 = _sc_gather(h1, cs1.reshape(NW * NCH, GW))

    out16, logits2 = _w2(rows1, gate1, W2, b2.reshape(1, H), Wg2)

    ss2, cs2, gate2 = _routing(logits2)
    disp2 = _sc_scatter(out16, ss2.reshape(NW * NCH, GW))
    h2 = _ffn(disp2, We2a, be2a.reshape(E, 1, H),
              We2b, be2b.reshape(E, 1, H))
    rows2 = _sc_gather(h2, cs2.reshape(NW * NCH, GW))

    loss = _final(hidden16, rows2, gate2, W3,
                  b3.reshape(1, H), y.reshape(B, 1).astype(I32))
    return loss.reshape(())
